# baseline (speedup 1.0000x reference)
import functools

import jax
import jax.numpy as jnp
from jax.experimental import pallas as pl
from jax.experimental.pallas import tpu as pltpu

_SLOPE = 0.01
F32 = jnp.float32
BF16 = jnp.bfloat16
_VMEM = 100 * 1024 * 1024


def _lk(y):
    return jnp.maximum(y, _SLOPE * y)


def _template():
    tnt = lambda: {k: 0 for k in ("w1", "s1", "wb", "s2", "w3", "s3",
                                  "wr", "sr")}
    d = {"stage0": {"w": 0, "s": 0}}
    for n in ("stage1_1", "stage1_2", "stage1_3", "stage1_4",
              "stage2_1", "stage2_2", "stage2_3", "stage2_4",
              "stage3_1", "stage3_2", "stage3_3", "stage3_4", "stage4"):
        d[n] = [tnt(), tnt(), tnt()]
    for n in ("stage1_", "stage2_", "stage3_"):
        d[n] = {"w": 0, "s": 0}
    for n in ("layer1_path", "layer2_path", "layer3_path"):
        d[n] = {"scale": 0, "shift": 0, "w": 0, "b": 0}
    d["fc1"] = {"w": 0, "b": 0}
    d["fc2"] = {"w": 0, "b": 0}
    return d


_TREEDEF = jax.tree_util.tree_flatten((_template(), 0))[1]


def _floor_body(*refs, L, width, tile, nt, exact):
    x_ref = refs[0]
    o_ref = refs[1 + 8 * nt]
    hbuf = refs[2 + 8 * nt]
    M = x_ref.shape[0]
    row = jax.lax.broadcasted_iota(jnp.int32, (M, 1), 0)
    pos = jnp.bitwise_and(row, L - 1)
    first = pos == 0
    last = pos == L - 1
    zrow = jnp.zeros((M, width), BF16)

    h = x_ref[...]
    for t in range(nt):
        w1, s1, wb, s2, w3, s3, wr, sr = refs[1 + 8 * t: 9 + 8 * t]
        h1 = _lk(jnp.dot(h, w1[...], preferred_element_type=F32)
                 + s1[...]).astype(BF16)
        hbuf[pl.ds(8, M), :] = h1
        hp = jnp.where(first, zrow, hbuf[pl.ds(7, M), :])
        hn = jnp.where(last, zrow, hbuf[pl.ds(9, M), :])

        Cout = o_ref.shape[1]
        nc = 128 if exact else Cout
        wbv, s2v, w3v, wrv, srv, s3v = (wb[...], s2[...], w3[...],
                                        wr[...], sr[...], s3[...])
        accs = [jnp.zeros((M, nc), F32) for _ in range(Cout // nc)]
        for m in range(width // tile):
            lo = m * tile
            g = (jnp.dot(hp[:, lo:lo + tile], wbv[0, lo:lo + tile, :],
                         preferred_element_type=F32)
                 + jnp.dot(h1[:, lo:lo + tile], wbv[1, lo:lo + tile, :],
                           preferred_element_type=F32)
                 + jnp.dot(hn[:, lo:lo + tile], wbv[2, lo:lo + tile, :],
                           preferred_element_type=F32))
            h2 = _lk(g + s2v[:, lo:lo + tile]).astype(BF16)
            for half in range(tile // 128):
                hl = half * 128
                for c in range(Cout // nc):
                    accs[c] = accs[c] + jnp.dot(
                        h2[:, hl:hl + 128],
                        w3v[lo + hl:lo + hl + 128, c * nc:(c + 1) * nc],
                        preferred_element_type=F32)
        outs = []
        for c in range(Cout // nc):
            cs = slice(c * nc, (c + 1) * nc)
            idy = (jnp.dot(h, wrv[:, cs], preferred_element_type=F32)
                   + srv[:, cs])
            outs.append(_lk(accs[c] + s3v[:, cs] + idy).astype(BF16))
        h = outs[0] if len(outs) == 1 else jnp.concatenate(outs, axis=1)
    o_ref[...] = h


def _pack_wb(wb, tile):
    if tile == 128:
        return wb
    _, width, _ = wb.shape
    nt2 = width // 256
    d = wb.reshape(3, nt2, 2, 128, 128)
    z = jnp.zeros((3, nt2, 128, 128), wb.dtype)
    top = jnp.concatenate([d[:, :, 0], z], axis=-1)
    bot = jnp.concatenate([z, d[:, :, 1]], axis=-1)
    return jnp.concatenate([top, bot], axis=2).reshape(3, width, 256)


def _run_floor(x3, tps, exact=False):
    B, L, Cin = x3.shape
    x2d = x3.reshape(B * L, Cin)
    M = B * L
    width = tps[0]["wb"].shape[1]
    Cout = tps[0]["w3"].shape[1]
    tile = 128 if exact else (256 if width % 256 == 0 else 128)
    if exact:
        Mc = M
    else:
        Mc = M // 2 if M >= 512 else M
    nt = len(tps)

    args = [x2d]
    in_specs = [pl.BlockSpec((Mc, Cin), lambda n: (n, 0))]
    flops = 0
    for tp in tps:
        cin_t = tp["w1"].shape[0]
        args += [tp["w1"], tp["s1"], _pack_wb(tp["wb"], tile), tp["s2"],
                 tp["w3"], tp["s3"], tp["wr"], tp["sr"]]
        in_specs += [
            pl.BlockSpec((cin_t, width), lambda n: (0, 0)),
            pl.BlockSpec((1, width), lambda n: (0, 0)),
            pl.BlockSpec((3, width, tile), lambda n: (0, 0, 0)),
            pl.BlockSpec((1, width), lambda n: (0, 0)),
            pl.BlockSpec((width, Cout), lambda n: (0, 0)),
            pl.BlockSpec((1, Cout), lambda n: (0, 0)),
            pl.BlockSpec((cin_t, Cout), lambda n: (0, 0)),
            pl.BlockSpec((1, Cout), lambda n: (0, 0)),
        ]
        flops += 2 * M * (cin_t * width + 3 * tile * width
                          + width * Cout + cin_t * Cout)
    bytes_acc = sum(int(a.size) * a.dtype.itemsize for a in args) \
        + M * Cout * 2

    out = pl.pallas_call(
        functools.partial(_floor_body, L=L, width=width, tile=tile, nt=nt,
                          exact=exact),
        out_shape=jax.ShapeDtypeStruct((M, Cout), BF16),
        grid_spec=pltpu.PrefetchScalarGridSpec(
            num_scalar_prefetch=0,
            grid=(M // Mc,),
            in_specs=in_specs,
            out_specs=pl.BlockSpec((Mc, Cout), lambda n: (n, 0)),
            scratch_shapes=[pltpu.VMEM((Mc + 16, width), BF16)],
        ),
        compiler_params=pltpu.CompilerParams(
            dimension_semantics=("parallel",),
            vmem_limit_bytes=_VMEM),
        cost_estimate=pl.CostEstimate(flops=flops, transcendentals=0,
                                      bytes_accessed=bytes_acc),
    )(*args)
    return out.reshape(B, L, Cout)


def _pre_body(p0, p1, p2, w0, s0, q1, wi1, si1, q2, wi2, si2, q3, wi3, si3,
              o0, o1, o2, o3):
    w, s = w0[...], s0[...]
    y = None
    for p_ref in (p0, p1, p2):
        a = _lk(jnp.dot(p_ref[...], w, preferred_element_type=F32) + s)
        y = a if y is None else jnp.maximum(y, a)
    o0[...] = y.astype(BF16)
    for q, wi, si, o in ((q1, wi1, si1, o1), (q2, wi2, si2, o2),
                         (q3, wi3, si3, o3)):
        o[...] = (jnp.dot(q[...], wi[...], preferred_element_type=F32)
                  + si[...]).astype(BF16)


def _run_pre(parts, p0, q1, p1, q2, p2, q3, p3, B):
    shapes = (jax.ShapeDtypeStruct((B * 64, 64), BF16),
              jax.ShapeDtypeStruct((B * 64, 256), BF16),
              jax.ShapeDtypeStruct((B * 32, 512), BF16),
              jax.ShapeDtypeStruct((B * 16, 1024), BF16))
    args = (parts[0], parts[1], parts[2], p0["w"], p0["s"],
            q1, p1["w"], p1["s"], q2, p2["w"], p2["s"], q3, p3["w"], p3["s"])
    in_specs = []
    for a in args:
        if a.shape[0] in (1, 8):
            in_specs.append(pl.BlockSpec(a.shape, lambda n: (0, 0)))
        else:
            in_specs.append(pl.BlockSpec((a.shape[0] // 2, a.shape[1]),
                                         lambda n: (n, 0)))
    out_specs = [pl.BlockSpec((s.shape[0] // 2, s.shape[1]),
                              lambda n: (n, 0)) for s in shapes]
    return pl.pallas_call(
        _pre_body,
        out_shape=tuple(shapes),
        grid_spec=pltpu.PrefetchScalarGridSpec(
            num_scalar_prefetch=0, grid=(2,),
            in_specs=in_specs, out_specs=out_specs),
        compiler_params=pltpu.CompilerParams(
            dimension_semantics=("parallel",),
            vmem_limit_bytes=_VMEM),
    )(*args)


def _path_body(h_ref, sc_ref, sh_ref, w_ref, b_ref, o_ref, *, C):
    hv = h_ref[...].astype(F32)
    pooled = 0.5 * (hv[:, :C] + hv[:, C:])
    a = _lk(pooled * sc_ref[...] + sh_ref[...]).astype(BF16)
    o_ref[...] = (jnp.dot(a, w_ref[...], preferred_element_type=F32)
                  + b_ref[...]).astype(BF16)


def _run_path(hcat, pp):
    B, L2, C = hcat.shape
    pairs = hcat.reshape(B, L2 // 2, 2 * C)[:, ::2]
    Lo = pairs.shape[1]
    M = B * Lo
    h2 = pairs.reshape(M, 2 * C)
    out = pl.pallas_call(
        functools.partial(_path_body, C=C),
        out_shape=jax.ShapeDtypeStruct((M, C), BF16),
        grid_spec=pltpu.PrefetchScalarGridSpec(
            num_scalar_prefetch=0, grid=(2,),
            in_specs=[
                pl.BlockSpec((M // 2, 2 * C), lambda n: (n, 0)),
                pl.BlockSpec((1, C), lambda n: (0, 0)),
                pl.BlockSpec((1, C), lambda n: (0, 0)),
                pl.BlockSpec((C, C), lambda n: (0, 0)),
                pl.BlockSpec((1, C), lambda n: (0, 0)),
            ],
            out_specs=pl.BlockSpec((M // 2, C), lambda n: (n, 0))),
        compiler_params=pltpu.CompilerParams(
            dimension_semantics=("parallel",),
            vmem_limit_bytes=_VMEM),
    )(h2, pp["scale"], pp["shift"], pp["w"], pp["b"])
    return out.reshape(B, Lo, C)


def _head_body(h_ref, w1_ref, b1_ref, w2_ref, b2_ref, o_ref):
    pooled = jnp.mean(h_ref[...].astype(F32), axis=1)
    a = _lk(jnp.dot(pooled.astype(BF16), w1_ref[...],
                    preferred_element_type=F32) + b1_ref[...])
    o_ref[...] = jnp.dot(a.astype(BF16), w2_ref[...],
                         preferred_element_type=F32) + b2_ref[...]


def _run_head(h, fc1, fc2):
    B = h.shape[0]
    label = fc2["w"].shape[1]
    return pl.pallas_call(
        _head_body,
        out_shape=jax.ShapeDtypeStruct((B, label), F32),
        in_specs=[pl.BlockSpec(memory_space=pltpu.MemorySpace.VMEM)] * 5,
        out_specs=pl.BlockSpec(memory_space=pltpu.MemorySpace.VMEM),
    )(h, fc1["w"], fc1["b"], fc2["w"], fc2["b"])


def _forward(p, x):
    B = x.shape[0]
    xs = x[:, :, 0]
    taps = jnp.stack([xs[:, t:t + 192] for t in range(7)], axis=-1)
    taps8 = jnp.pad(taps, ((0, 0), (0, 0), (0, 1)))
    parts = [taps8[:, j::3, :].reshape(B * 64, 8).astype(BF16)
             for j in range(3)]
    pat = jnp.concatenate([taps, jnp.ones((B, 192, 1), F32)], axis=-1)
    q1 = pat.reshape(B * 64, 3, 8).mean(axis=1)
    q2 = pat.reshape(B * 32, 6, 8).mean(axis=1)
    xs3 = jnp.pad(xs, ((0, 0), (2, 2)))
    taps3 = jnp.stack([xs3[:, t:t + 196] for t in range(7)], axis=-1)
    pat3 = jnp.pad(jnp.concatenate([taps3, jnp.ones((B, 196, 1), F32)],
                                   axis=-1), ((0, 0), (2, 2), (0, 0)))
    q3 = pat3[:, :192, :].reshape(B * 16, 12, 8).mean(axis=1)

    h0, id1f, id2f, id3f = _run_pre(parts, p["stage0"], q1, p["stage1_"],
                                    q2, p["stage2_"], q3, p["stage3_"], B)
    h = h0.reshape(B, 64, 64)
    id1 = id1f.reshape(B, 64, 256)
    id2 = id2f.reshape(B, 32, 512)
    id3 = id3f.reshape(B, 16, 1024)
    id1e, id2e, id3e = id1[:, ::2], id2[:, ::2], id3[:, ::2]

    h = _run_floor(h, p["stage1_1"], exact=True)
    for nm in ("stage1_2", "stage1_3", "stage1_4"):
        h = _run_floor(jnp.concatenate([h[:, ::2], id1e], axis=1), p[nm],
                       exact=True)
    h = _run_path(jnp.concatenate([h, id1], axis=1), p["layer1_path"])

    h = _run_floor(h, p["stage2_1"])
    for nm in ("stage2_2", "stage2_3", "stage2_4"):
        h = _run_floor(jnp.concatenate([h[:, ::2], id2e], axis=1), p[nm])
    h = _run_path(jnp.concatenate([h, id2], axis=1), p["layer2_path"])

    h = _run_floor(h, p["stage3_1"])
    for nm in ("stage3_2", "stage3_3", "stage3_4"):
        h = _run_floor(jnp.concatenate([h[:, ::2], id3e], axis=1), p[nm])
    h = _run_path(jnp.concatenate([h, id3], axis=1), p["layer3_path"])

    h = _run_floor(h, p["stage4"])
    return _run_head(h, p["fc1"], p["fc2"])


def kernel(arr0000, arr0001, arr0002, arr0003, arr0004, arr0005, arr0006, arr0007, arr0008, arr0009, arr0010, arr0011, arr0012, arr0013, arr0014, arr0015, arr0016, arr0017, arr0018, arr0019, arr0020, arr0021, arr0022, arr0023, arr0024, arr0025, arr0026, arr0027, arr0028, arr0029, arr0030, arr0031, arr0032, arr0033, arr0034, arr0035, arr0036, arr0037, arr0038, arr0039, arr0040, arr0041, arr0042, arr0043, arr0044, arr0045, arr0046, arr0047, arr0048, arr0049, arr0050, arr0051, arr0052, arr0053, arr0054, arr0055, arr0056, arr0057, arr0058, arr0059, arr0060, arr0061, arr0062, arr0063, arr0064, arr0065, arr0066, arr0067, arr0068, arr0069, arr0070, arr0071, arr0072, arr0073, arr0074, arr0075, arr0076, arr0077, arr0078, arr0079, arr0080, arr0081, arr0082, arr0083, arr0084, arr0085, arr0086, arr0087, arr0088, arr0089, arr0090, arr0091, arr0092, arr0093, arr0094, arr0095, arr0096, arr0097, arr0098, arr0099, arr0100, arr0101, arr0102, arr0103, arr0104, arr0105, arr0106, arr0107, arr0108, arr0109, arr0110, arr0111, arr0112, arr0113, arr0114, arr0115, arr0116, arr0117, arr0118, arr0119, arr0120, arr0121, arr0122, arr0123, arr0124, arr0125, arr0126, arr0127, arr0128, arr0129, arr0130, arr0131, arr0132, arr0133, arr0134, arr0135, arr0136, arr0137, arr0138, arr0139, arr0140, arr0141, arr0142, arr0143, arr0144, arr0145, arr0146, arr0147, arr0148, arr0149, arr0150, arr0151, arr0152, arr0153, arr0154, arr0155, arr0156, arr0157, arr0158, arr0159, arr0160, arr0161, arr0162, arr0163, arr0164, arr0165, arr0166, arr0167, arr0168, arr0169, arr0170, arr0171, arr0172, arr0173, arr0174, arr0175, arr0176, arr0177, arr0178, arr0179, arr0180, arr0181, arr0182, arr0183, arr0184, arr0185, arr0186, arr0187, arr0188, arr0189, arr0190, arr0191, arr0192, arr0193, arr0194, arr0195, arr0196, arr0197, arr0198, arr0199, arr0200, arr0201, arr0202, arr0203, arr0204, arr0205, arr0206, arr0207, arr0208, arr0209, arr0210, arr0211, arr0212, arr0213, arr0214, arr0215, arr0216, arr0217, arr0218, arr0219, arr0220, arr0221, arr0222, arr0223, arr0224, arr0225, arr0226, arr0227, arr0228, arr0229, arr0230, arr0231, arr0232, arr0233, arr0234, arr0235, arr0236, arr0237, arr0238, arr0239, arr0240, arr0241, arr0242, arr0243, arr0244, arr0245, arr0246, arr0247, arr0248, arr0249, arr0250, arr0251, arr0252, arr0253, arr0254, arr0255, arr0256, arr0257, arr0258, arr0259, arr0260, arr0261, arr0262, arr0263, arr0264, arr0265, arr0266, arr0267, arr0268, arr0269, arr0270, arr0271, arr0272, arr0273, arr0274, arr0275, arr0276, arr0277, arr0278, arr0279, arr0280, arr0281, arr0282, arr0283, arr0284, arr0285, arr0286, arr0287, arr0288, arr0289, arr0290, arr0291, arr0292, arr0293, arr0294, arr0295, arr0296, arr0297, arr0298, arr0299, arr0300, arr0301, arr0302, arr0303, arr0304, arr0305, arr0306, arr0307, arr0308, arr0309, arr0310, arr0311, arr0312, arr0313, arr0314, arr0315, arr0316, arr0317, arr0318, arr0319, arr0320, arr0321, arr0322, arr0323, arr0324, arr0325, arr0326, arr0327, arr0328, arr0329, arr0330, arr0331, arr0332, arr0333, arr0334, arr0335, arr0336):
    flat = list(locals().values())
    params, x = jax.tree_util.tree_unflatten(_TREEDEF, flat)
    return _forward(params, x)


# speedup vs baseline: 1.0047x; 1.0047x over previous
import functools

import jax
import jax.numpy as jnp
from jax.experimental import pallas as pl
from jax.experimental.pallas import tpu as pltpu

_SLOPE = 0.01
F32 = jnp.float32
BF16 = jnp.bfloat16
_VMEM = 100 * 1024 * 1024


def _lk(y):
    return jnp.maximum(y, _SLOPE * y)


def _template():
    tnt = lambda: {k: 0 for k in ("w1", "s1", "wb", "s2", "w3", "s3",
                                  "wr", "sr")}
    d = {"stage0": {"w": 0, "s": 0}}
    for n in ("stage1_1", "stage1_2", "stage1_3", "stage1_4",
              "stage2_1", "stage2_2", "stage2_3", "stage2_4",
              "stage3_1", "stage3_2", "stage3_3", "stage3_4", "stage4"):
        d[n] = [tnt(), tnt(), tnt()]
    for n in ("stage1_", "stage2_", "stage3_"):
        d[n] = {"w": 0, "s": 0}
    for n in ("layer1_path", "layer2_path", "layer3_path"):
        d[n] = {"scale": 0, "shift": 0, "w": 0, "b": 0}
    d["fc1"] = {"w": 0, "b": 0}
    d["fc2"] = {"w": 0, "b": 0}
    return d


_TREEDEF = jax.tree_util.tree_flatten((_template(), 0))[1]


def _floor_body(*refs, L, width, tile, nt, exact):
    x_ref = refs[0]
    o_ref = refs[1 + 8 * nt]
    hbuf = refs[2 + 8 * nt]
    M = x_ref.shape[0]
    row = jax.lax.broadcasted_iota(jnp.int32, (M, 1), 0)
    pos = jnp.bitwise_and(row, L - 1)
    first = pos == 0
    last = pos == L - 1
    zrow = jnp.zeros((M, width), BF16)

    h = x_ref[...]
    for t in range(nt):
        w1, s1, wb, s2, w3, s3, wr, sr = refs[1 + 8 * t: 9 + 8 * t]
        h1 = _lk(jnp.dot(h, w1[...], preferred_element_type=F32)
                 + s1[...]).astype(BF16)
        hbuf[pl.ds(8, M), :] = h1
        hp = jnp.where(first, zrow, hbuf[pl.ds(7, M), :])
        hn = jnp.where(last, zrow, hbuf[pl.ds(9, M), :])

        Cout = o_ref.shape[1]
        nc = min(512, Cout // 2) if exact else Cout
        wbv, s2v, w3v, wrv, srv, s3v = (wb[...], s2[...], w3[...],
                                        wr[...], sr[...], s3[...])
        accs = [jnp.zeros((M, nc), F32) for _ in range(Cout // nc)]
        for m in range(width // tile):
            lo = m * tile
            g = (jnp.dot(hp[:, lo:lo + tile], wbv[0, lo:lo + tile, :],
                         preferred_element_type=F32)
                 + jnp.dot(h1[:, lo:lo + tile], wbv[1, lo:lo + tile, :],
                           preferred_element_type=F32)
                 + jnp.dot(hn[:, lo:lo + tile], wbv[2, lo:lo + tile, :],
                           preferred_element_type=F32))
            h2 = _lk(g + s2v[:, lo:lo + tile]).astype(BF16)
            for half in range(tile // 128):
                hl = half * 128
                for c in range(Cout // nc):
                    accs[c] = accs[c] + jnp.dot(
                        h2[:, hl:hl + 128],
                        w3v[lo + hl:lo + hl + 128, c * nc:(c + 1) * nc],
                        preferred_element_type=F32)
        outs = []
        for c in range(Cout // nc):
            cs = slice(c * nc, (c + 1) * nc)
            idy = (jnp.dot(h, wrv[:, cs], preferred_element_type=F32)
                   + srv[:, cs])
            outs.append(_lk(accs[c] + s3v[:, cs] + idy).astype(BF16))
        h = outs[0] if len(outs) == 1 else jnp.concatenate(outs, axis=1)
    o_ref[...] = h


def _pack_wb(wb, tile):
    if tile == 128:
        return wb
    _, width, _ = wb.shape
    nt2 = width // 256
    d = wb.reshape(3, nt2, 2, 128, 128)
    z = jnp.zeros((3, nt2, 128, 128), wb.dtype)
    top = jnp.concatenate([d[:, :, 0], z], axis=-1)
    bot = jnp.concatenate([z, d[:, :, 1]], axis=-1)
    return jnp.concatenate([top, bot], axis=2).reshape(3, width, 256)


def _run_floor(x3, tps, exact=False):
    B, L, Cin = x3.shape
    x2d = x3.reshape(B * L, Cin)
    M = B * L
    width = tps[0]["wb"].shape[1]
    Cout = tps[0]["w3"].shape[1]
    tile = 128 if exact else (256 if width % 256 == 0 else 128)
    if exact:
        Mc = M
    else:
        Mc = M // 2 if M >= 512 else M
    nt = len(tps)

    args = [x2d]
    in_specs = [pl.BlockSpec((Mc, Cin), lambda n: (n, 0))]
    flops = 0
    for tp in tps:
        cin_t = tp["w1"].shape[0]
        args += [tp["w1"], tp["s1"], _pack_wb(tp["wb"], tile), tp["s2"],
                 tp["w3"], tp["s3"], tp["wr"], tp["sr"]]
        in_specs += [
            pl.BlockSpec((cin_t, width), lambda n: (0, 0)),
            pl.BlockSpec((1, width), lambda n: (0, 0)),
            pl.BlockSpec((3, width, tile), lambda n: (0, 0, 0)),
            pl.BlockSpec((1, width), lambda n: (0, 0)),
            pl.BlockSpec((width, Cout), lambda n: (0, 0)),
            pl.BlockSpec((1, Cout), lambda n: (0, 0)),
            pl.BlockSpec((cin_t, Cout), lambda n: (0, 0)),
            pl.BlockSpec((1, Cout), lambda n: (0, 0)),
        ]
        flops += 2 * M * (cin_t * width + 3 * tile * width
                          + width * Cout + cin_t * Cout)
    bytes_acc = sum(int(a.size) * a.dtype.itemsize for a in args) \
        + M * Cout * 2

    out = pl.pallas_call(
        functools.partial(_floor_body, L=L, width=width, tile=tile, nt=nt,
                          exact=exact),
        out_shape=jax.ShapeDtypeStruct((M, Cout), BF16),
        grid_spec=pltpu.PrefetchScalarGridSpec(
            num_scalar_prefetch=0,
            grid=(M // Mc,),
            in_specs=in_specs,
            out_specs=pl.BlockSpec((Mc, Cout), lambda n: (n, 0)),
            scratch_shapes=[pltpu.VMEM((Mc + 16, width), BF16)],
        ),
        compiler_params=pltpu.CompilerParams(
            dimension_semantics=("parallel",),
            vmem_limit_bytes=_VMEM),
        cost_estimate=pl.CostEstimate(flops=flops, transcendentals=0,
                                      bytes_accessed=bytes_acc),
    )(*args)
    return out.reshape(B, L, Cout)


def _seed_where_lk(y):
    return jnp.where(y > 0, y, _SLOPE * y)


def _seed_tenant_kernel(x_ref, w1_ref, s1_ref, wb_ref, s2_ref, w3_ref,
                        s3_ref, wr_ref, sr_ref, o_ref, hbuf_ref, *, L, width):
    M = x_ref.shape[0]
    x = x_ref[...]
    h1 = _seed_where_lk(jnp.dot(x, w1_ref[...], preferred_element_type=F32)
                        + s1_ref[...]).astype(BF16)
    hbuf_ref[...] = jnp.zeros_like(hbuf_ref)
    hbuf_ref[pl.ds(8, M), :] = h1
    row = jax.lax.broadcasted_iota(jnp.int32, (M, 1), 0)
    is_first = row == 0
    is_last = row == L - 1
    for b in range(1, M // L):
        is_first = jnp.logical_or(is_first, row == b * L)
        is_last = jnp.logical_or(is_last, row == b * L + L - 1)
    zeros = jnp.zeros((M, width), BF16)
    h_prev = jnp.where(is_first, zeros, hbuf_ref[pl.ds(7, M), :])
    h_next = jnp.where(is_last, zeros, hbuf_ref[pl.ds(9, M), :])
    s2 = s2_ref[...]
    wb0, wb1, wb2 = wb_ref[0], wb_ref[1], wb_ref[2]
    tn = o_ref.shape[1]
    acc = jnp.zeros((M, tn), F32)
    for j in range(width // 128):
        lo = j * 128
        g = (jnp.dot(h_prev[:, lo:lo + 128], wb0[lo:lo + 128, :],
                     preferred_element_type=F32)
             + jnp.dot(h1[:, lo:lo + 128], wb1[lo:lo + 128, :],
                       preferred_element_type=F32)
             + jnp.dot(h_next[:, lo:lo + 128], wb2[lo:lo + 128, :],
                       preferred_element_type=F32))
        h2 = _seed_where_lk(g + s2[:, lo:lo + 128]).astype(BF16)
        acc = acc + jnp.dot(h2, w3_ref[lo:lo + 128, :],
                            preferred_element_type=F32)
    idy = jnp.dot(x, wr_ref[...], preferred_element_type=F32) + sr_ref[...]
    o_ref[...] = _seed_where_lk(acc + s3_ref[...] + idy).astype(o_ref.dtype)


def _seed_tenant(x2d, tp, L):
    M, Cin = x2d.shape
    width = tp["w1"].shape[1]
    Cout = tp["w3"].shape[1]
    tn = min(512, Cout // 2)
    nsteps = Cout // tn
    fn = functools.partial(_seed_tenant_kernel, L=L, width=width)
    flops = 2 * M * ((Cin + 3 * 128) * width * nsteps
                     + width * Cout + Cin * Cout)
    bytes_accessed = 2 * (M * Cin + Cin * width + 3 * width * 128
                          + width * Cout + Cin * Cout + M * Cout) \
        + 4 * (2 * width + 2 * Cout)
    return pl.pallas_call(
        fn,
        out_shape=jax.ShapeDtypeStruct((M, Cout), BF16),
        grid_spec=pltpu.PrefetchScalarGridSpec(
            num_scalar_prefetch=0,
            grid=(nsteps,),
            in_specs=[
                pl.BlockSpec((M, Cin), lambda n: (0, 0)),
                pl.BlockSpec((Cin, width), lambda n: (0, 0)),
                pl.BlockSpec((1, width), lambda n: (0, 0)),
                pl.BlockSpec((3, width, 128), lambda n: (0, 0, 0)),
                pl.BlockSpec((1, width), lambda n: (0, 0)),
                pl.BlockSpec((width, tn), lambda n: (0, n)),
                pl.BlockSpec((1, tn), lambda n: (0, n)),
                pl.BlockSpec((Cin, tn), lambda n: (0, n)),
                pl.BlockSpec((1, tn), lambda n: (0, n)),
            ],
            out_specs=pl.BlockSpec((M, tn), lambda n: (0, n)),
            scratch_shapes=[pltpu.VMEM((M + 16, width), BF16)],
        ),
        compiler_params=pltpu.CompilerParams(
            dimension_semantics=("parallel",),
            vmem_limit_bytes=32 * 1024 * 1024),
        cost_estimate=pl.CostEstimate(flops=flops, transcendentals=0,
                                      bytes_accessed=bytes_accessed),
    )(x2d, tp["w1"], tp["s1"], tp["wb"], tp["s2"],
      tp["w3"], tp["s3"], tp["wr"], tp["sr"])


def _seed_floor(x3, tps):
    B, L, Cin = x3.shape
    h = x3.reshape(B * L, Cin)
    for tp in tps:
        h = _seed_tenant(h, tp, L)
    return h.reshape(B, L, -1)


def _seed_path_kernel(h_ref, sc_ref, sh_ref, w_ref, b_ref, o_ref, *, C):
    h = h_ref[...].astype(F32)
    pooled = 0.5 * (h[:, :C] + h[:, C:])
    a = _seed_where_lk(pooled * sc_ref[...] + sh_ref[...]).astype(BF16)
    y = jnp.dot(a, w_ref[...], preferred_element_type=F32) + b_ref[...]
    o_ref[...] = y.astype(o_ref.dtype)


def _seed_path(x3, pp):
    B, L, C = x3.shape
    Lo = L // 2
    h2 = x3[:, :Lo * 2, :].reshape(B * Lo, 2 * C)
    M = B * Lo
    tn = max(128, C // 2)
    out = pl.pallas_call(
        functools.partial(_seed_path_kernel, C=C),
        out_shape=jax.ShapeDtypeStruct((M, C), BF16),
        grid_spec=pltpu.PrefetchScalarGridSpec(
            num_scalar_prefetch=0,
            grid=(C // tn,),
            in_specs=[
                pl.BlockSpec((M, 2 * C), lambda n: (0, 0)),
                pl.BlockSpec((1, C), lambda n: (0, 0)),
                pl.BlockSpec((1, C), lambda n: (0, 0)),
                pl.BlockSpec((C, tn), lambda n: (0, n)),
                pl.BlockSpec((1, tn), lambda n: (0, n)),
            ],
            out_specs=pl.BlockSpec((M, tn), lambda n: (0, n))),
        compiler_params=pltpu.CompilerParams(
            dimension_semantics=("parallel",),
            vmem_limit_bytes=32 * 1024 * 1024),
    )(h2, pp["scale"], pp["shift"], pp["w"], pp["b"])
    return out.reshape(B, Lo, C)


def _pre_body(p0, p1, p2, w0, s0, q1, wi1, si1, q2, wi2, si2, q3, wi3, si3,
              o0, o1, o2, o3):
    w, s = w0[...], s0[...]
    y = None
    for p_ref in (p0, p1, p2):
        a = _lk(jnp.dot(p_ref[...], w, preferred_element_type=F32) + s)
        y = a if y is None else jnp.maximum(y, a)
    o0[...] = y.astype(BF16)
    for q, wi, si, o in ((q1, wi1, si1, o1), (q2, wi2, si2, o2),
                         (q3, wi3, si3, o3)):
        o[...] = (jnp.dot(q[...], wi[...], preferred_element_type=F32)
                  + si[...]).astype(BF16)


def _run_pre(parts, p0, q1, p1, q2, p2, q3, p3, B):
    shapes = (jax.ShapeDtypeStruct((B * 64, 64), BF16),
              jax.ShapeDtypeStruct((B * 64, 256), BF16),
              jax.ShapeDtypeStruct((B * 32, 512), BF16),
              jax.ShapeDtypeStruct((B * 16, 1024), BF16))
    args = (parts[0], parts[1], parts[2], p0["w"], p0["s"],
            q1, p1["w"], p1["s"], q2, p2["w"], p2["s"], q3, p3["w"], p3["s"])
    in_specs = []
    for a in args:
        if a.shape[0] in (1, 8):
            in_specs.append(pl.BlockSpec(a.shape, lambda n: (0, 0)))
        else:
            in_specs.append(pl.BlockSpec((a.shape[0] // 2, a.shape[1]),
                                         lambda n: (n, 0)))
    out_specs = [pl.BlockSpec((s.shape[0] // 2, s.shape[1]),
                              lambda n: (n, 0)) for s in shapes]
    return pl.pallas_call(
        _pre_body,
        out_shape=tuple(shapes),
        grid_spec=pltpu.PrefetchScalarGridSpec(
            num_scalar_prefetch=0, grid=(2,),
            in_specs=in_specs, out_specs=out_specs),
        compiler_params=pltpu.CompilerParams(
            dimension_semantics=("parallel",),
            vmem_limit_bytes=_VMEM),
    )(*args)


def _path_body(h_ref, sc_ref, sh_ref, w_ref, b_ref, o_ref, *, C):
    hv = h_ref[...].astype(F32)
    pooled = 0.5 * (hv[:, :C] + hv[:, C:])
    a = _lk(pooled * sc_ref[...] + sh_ref[...]).astype(BF16)
    o_ref[...] = (jnp.dot(a, w_ref[...], preferred_element_type=F32)
                  + b_ref[...]).astype(BF16)


def _run_path(hcat, pp):
    B, L2, C = hcat.shape
    pairs = hcat.reshape(B, L2 // 2, 2 * C)[:, ::2]
    Lo = pairs.shape[1]
    M = B * Lo
    h2 = pairs.reshape(M, 2 * C)
    out = pl.pallas_call(
        functools.partial(_path_body, C=C),
        out_shape=jax.ShapeDtypeStruct((M, C), BF16),
        grid_spec=pltpu.PrefetchScalarGridSpec(
            num_scalar_prefetch=0, grid=(2,),
            in_specs=[
                pl.BlockSpec((M // 2, 2 * C), lambda n: (n, 0)),
                pl.BlockSpec((1, C), lambda n: (0, 0)),
                pl.BlockSpec((1, C), lambda n: (0, 0)),
                pl.BlockSpec((C, C), lambda n: (0, 0)),
                pl.BlockSpec((1, C), lambda n: (0, 0)),
            ],
            out_specs=pl.BlockSpec((M // 2, C), lambda n: (n, 0))),
        compiler_params=pltpu.CompilerParams(
            dimension_semantics=("parallel",),
            vmem_limit_bytes=_VMEM),
    )(h2, pp["scale"], pp["shift"], pp["w"], pp["b"])
    return out.reshape(B, Lo, C)


def _head_body(h_ref, w1_ref, b1_ref, w2_ref, b2_ref, o_ref):
    pooled = jnp.mean(h_ref[...].astype(F32), axis=1)
    a = _lk(jnp.dot(pooled.astype(BF16), w1_ref[...],
                    preferred_element_type=F32) + b1_ref[...])
    o_ref[...] = jnp.dot(a.astype(BF16), w2_ref[...],
                         preferred_element_type=F32) + b2_ref[...]


def _run_head(h, fc1, fc2):
    B = h.shape[0]
    label = fc2["w"].shape[1]
    return pl.pallas_call(
        _head_body,
        out_shape=jax.ShapeDtypeStruct((B, label), F32),
        in_specs=[pl.BlockSpec(memory_space=pltpu.MemorySpace.VMEM)] * 5,
        out_specs=pl.BlockSpec(memory_space=pltpu.MemorySpace.VMEM),
    )(h, fc1["w"], fc1["b"], fc2["w"], fc2["b"])


def _forward(p, x):
    B = x.shape[0]
    xs = x[:, :, 0]
    taps = jnp.stack([xs[:, t:t + 192] for t in range(7)], axis=-1)
    taps8 = jnp.pad(taps, ((0, 0), (0, 0), (0, 1)))
    parts = [taps8[:, j::3, :].reshape(B * 64, 8).astype(BF16)
             for j in range(3)]
    pat = jnp.concatenate([taps, jnp.ones((B, 192, 1), F32)], axis=-1)
    q1 = pat.reshape(B * 64, 3, 8).mean(axis=1)
    q2 = pat.reshape(B * 32, 6, 8).mean(axis=1)
    xs3 = jnp.pad(xs, ((0, 0), (2, 2)))
    taps3 = jnp.stack([xs3[:, t:t + 196] for t in range(7)], axis=-1)
    pat3 = jnp.pad(jnp.concatenate([taps3, jnp.ones((B, 196, 1), F32)],
                                   axis=-1), ((0, 0), (2, 2), (0, 0)))
    q3 = pat3[:, :192, :].reshape(B * 16, 12, 8).mean(axis=1)

    h0, id1f, id2f, id3f = _run_pre(parts, p["stage0"], q1, p["stage1_"],
                                    q2, p["stage2_"], q3, p["stage3_"], B)
    h = h0.reshape(B, 64, 64)
    id1 = id1f.reshape(B, 64, 256)
    id2 = id2f.reshape(B, 32, 512)
    id3 = id3f.reshape(B, 16, 1024)
    id1e, id2e, id3e = id1[:, ::2], id2[:, ::2], id3[:, ::2]

    h = _run_floor(h, p["stage1_1"], exact=True)
    for nm in ("stage1_2", "stage1_3", "stage1_4"):
        h = _run_floor(jnp.concatenate([h[:, ::2], id1e], axis=1), p[nm],
                       exact=True)
    h = _seed_path(jnp.concatenate([h, id1], axis=1), p["layer1_path"])

    h = _seed_floor(h[:, ::2], p["stage2_1"])
    for nm in ("stage2_2", "stage2_3", "stage2_4"):
        h = _seed_floor(jnp.concatenate([h[:, ::2], id2e], axis=1), p[nm])
    h = _seed_path(jnp.concatenate([h, id2], axis=1), p["layer2_path"])

    h = _seed_floor(h[:, ::2], p["stage3_1"])
    for nm in ("stage3_2", "stage3_3", "stage3_4"):
        h = _seed_floor(jnp.concatenate([h[:, ::2], id3e], axis=1), p[nm])
    h = _seed_path(jnp.concatenate([h, id3], axis=1), p["layer3_path"])

    h = _seed_floor(h[:, ::2], p["stage4"])
    return _run_head(h, p["fc1"], p["fc2"])


def kernel(arr0000, arr0001, arr0002, arr0003, arr0004, arr0005, arr0006, arr0007, arr0008, arr0009, arr0010, arr0011, arr0012, arr0013, arr0014, arr0015, arr0016, arr0017, arr0018, arr0019, arr0020, arr0021, arr0022, arr0023, arr0024, arr0025, arr0026, arr0027, arr0028, arr0029, arr0030, arr0031, arr0032, arr0033, arr0034, arr0035, arr0036, arr0037, arr0038, arr0039, arr0040, arr0041, arr0042, arr0043, arr0044, arr0045, arr0046, arr0047, arr0048, arr0049, arr0050, arr0051, arr0052, arr0053, arr0054, arr0055, arr0056, arr0057, arr0058, arr0059, arr0060, arr0061, arr0062, arr0063, arr0064, arr0065, arr0066, arr0067, arr0068, arr0069, arr0070, arr0071, arr0072, arr0073, arr0074, arr0075, arr0076, arr0077, arr0078, arr0079, arr0080, arr0081, arr0082, arr0083, arr0084, arr0085, arr0086, arr0087, arr0088, arr0089, arr0090, arr0091, arr0092, arr0093, arr0094, arr0095, arr0096, arr0097, arr0098, arr0099, arr0100, arr0101, arr0102, arr0103, arr0104, arr0105, arr0106, arr0107, arr0108, arr0109, arr0110, arr0111, arr0112, arr0113, arr0114, arr0115, arr0116, arr0117, arr0118, arr0119, arr0120, arr0121, arr0122, arr0123, arr0124, arr0125, arr0126, arr0127, arr0128, arr0129, arr0130, arr0131, arr0132, arr0133, arr0134, arr0135, arr0136, arr0137, arr0138, arr0139, arr0140, arr0141, arr0142, arr0143, arr0144, arr0145, arr0146, arr0147, arr0148, arr0149, arr0150, arr0151, arr0152, arr0153, arr0154, arr0155, arr0156, arr0157, arr0158, arr0159, arr0160, arr0161, arr0162, arr0163, arr0164, arr0165, arr0166, arr0167, arr0168, arr0169, arr0170, arr0171, arr0172, arr0173, arr0174, arr0175, arr0176, arr0177, arr0178, arr0179, arr0180, arr0181, arr0182, arr0183, arr0184, arr0185, arr0186, arr0187, arr0188, arr0189, arr0190, arr0191, arr0192, arr0193, arr0194, arr0195, arr0196, arr0197, arr0198, arr0199, arr0200, arr0201, arr0202, arr0203, arr0204, arr0205, arr0206, arr0207, arr0208, arr0209, arr0210, arr0211, arr0212, arr0213, arr0214, arr0215, arr0216, arr0217, arr0218, arr0219, arr0220, arr0221, arr0222, arr0223, arr0224, arr0225, arr0226, arr0227, arr0228, arr0229, arr0230, arr0231, arr0232, arr0233, arr0234, arr0235, arr0236, arr0237, arr0238, arr0239, arr0240, arr0241, arr0242, arr0243, arr0244, arr0245, arr0246, arr0247, arr0248, arr0249, arr0250, arr0251, arr0252, arr0253, arr0254, arr0255, arr0256, arr0257, arr0258, arr0259, arr0260, arr0261, arr0262, arr0263, arr0264, arr0265, arr0266, arr0267, arr0268, arr0269, arr0270, arr0271, arr0272, arr0273, arr0274, arr0275, arr0276, arr0277, arr0278, arr0279, arr0280, arr0281, arr0282, arr0283, arr0284, arr0285, arr0286, arr0287, arr0288, arr0289, arr0290, arr0291, arr0292, arr0293, arr0294, arr0295, arr0296, arr0297, arr0298, arr0299, arr0300, arr0301, arr0302, arr0303, arr0304, arr0305, arr0306, arr0307, arr0308, arr0309, arr0310, arr0311, arr0312, arr0313, arr0314, arr0315, arr0316, arr0317, arr0318, arr0319, arr0320, arr0321, arr0322, arr0323, arr0324, arr0325, arr0326, arr0327, arr0328, arr0329, arr0330, arr0331, arr0332, arr0333, arr0334, arr0335, arr0336):
    flat = list(locals().values())
    params, x = jax.tree_util.tree_unflatten(_TREEDEF, flat)
    return _forward(params, x)


# speedup vs baseline: 1.0666x; 1.0616x over previous
import functools

import jax
import jax.numpy as jnp
from jax.experimental import pallas as pl
from jax.experimental.pallas import tpu as pltpu

_SLOPE = 0.01
F32 = jnp.float32
BF16 = jnp.bfloat16
_VMEM = 100 * 1024 * 1024


def _lk(y):
    return jnp.maximum(y, _SLOPE * y)


def _template():
    tnt = lambda: {k: 0 for k in ("w1", "s1", "wb", "s2", "w3", "s3",
                                  "wr", "sr")}
    d = {"stage0": {"w": 0, "s": 0}}
    for n in ("stage1_1", "stage1_2", "stage1_3", "stage1_4",
              "stage2_1", "stage2_2", "stage2_3", "stage2_4",
              "stage3_1", "stage3_2", "stage3_3", "stage3_4", "stage4"):
        d[n] = [tnt(), tnt(), tnt()]
    for n in ("stage1_", "stage2_", "stage3_"):
        d[n] = {"w": 0, "s": 0}
    for n in ("layer1_path", "layer2_path", "layer3_path"):
        d[n] = {"scale": 0, "shift": 0, "w": 0, "b": 0}
    d["fc1"] = {"w": 0, "b": 0}
    d["fc2"] = {"w": 0, "b": 0}
    return d


_TREEDEF = jax.tree_util.tree_flatten((_template(), 0))[1]


def _floor_body(*refs, L, width, tile, nt, exact):
    x_ref = refs[0]
    o_ref = refs[1 + 8 * nt]
    hbuf = refs[2 + 8 * nt]
    M = x_ref.shape[0]
    row = jax.lax.broadcasted_iota(jnp.int32, (M, 1), 0)
    pos = jnp.bitwise_and(row, L - 1)
    first = pos == 0
    last = pos == L - 1
    zrow = jnp.zeros((M, width), BF16)

    h = x_ref[...]
    for t in range(nt):
        w1, s1, wb, s2, w3, s3, wr, sr = refs[1 + 8 * t: 9 + 8 * t]
        h1 = _lk(jnp.dot(h, w1[...], preferred_element_type=F32)
                 + s1[...]).astype(BF16)
        hbuf[pl.ds(8, M), :] = h1
        hp = jnp.where(first, zrow, hbuf[pl.ds(7, M), :])
        hn = jnp.where(last, zrow, hbuf[pl.ds(9, M), :])

        Cout = o_ref.shape[1]
        nc = min(512, Cout // 2) if exact else Cout
        wbv, s2v, w3v, wrv, srv, s3v = (wb[...], s2[...], w3[...],
                                        wr[...], sr[...], s3[...])
        accs = [jnp.zeros((M, nc), F32) for _ in range(Cout // nc)]
        for m in range(width // tile):
            lo = m * tile
            g = (jnp.dot(hp[:, lo:lo + tile], wbv[0, lo:lo + tile, :],
                         preferred_element_type=F32)
                 + jnp.dot(h1[:, lo:lo + tile], wbv[1, lo:lo + tile, :],
                           preferred_element_type=F32)
                 + jnp.dot(hn[:, lo:lo + tile], wbv[2, lo:lo + tile, :],
                           preferred_element_type=F32))
            h2 = _lk(g + s2v[:, lo:lo + tile]).astype(BF16)
            for half in range(tile // 128):
                hl = half * 128
                for c in range(Cout // nc):
                    accs[c] = accs[c] + jnp.dot(
                        h2[:, hl:hl + 128],
                        w3v[lo + hl:lo + hl + 128, c * nc:(c + 1) * nc],
                        preferred_element_type=F32)
        outs = []
        for c in range(Cout // nc):
            cs = slice(c * nc, (c + 1) * nc)
            idy = (jnp.dot(h, wrv[:, cs], preferred_element_type=F32)
                   + srv[:, cs])
            outs.append(_lk(accs[c] + s3v[:, cs] + idy).astype(BF16))
        h = outs[0] if len(outs) == 1 else jnp.concatenate(outs, axis=1)
    o_ref[...] = h


def _pack_wb(wb, tile):
    if tile == 128:
        return wb
    _, width, _ = wb.shape
    nt2 = width // 256
    d = wb.reshape(3, nt2, 2, 128, 128)
    z = jnp.zeros((3, nt2, 128, 128), wb.dtype)
    top = jnp.concatenate([d[:, :, 0], z], axis=-1)
    bot = jnp.concatenate([z, d[:, :, 1]], axis=-1)
    return jnp.concatenate([top, bot], axis=2).reshape(3, width, 256)


def _run_floor(x3, tps, exact=False):
    B, L, Cin = x3.shape
    x2d = x3.reshape(B * L, Cin)
    M = B * L
    width = tps[0]["wb"].shape[1]
    Cout = tps[0]["w3"].shape[1]
    tile = 128 if exact else (256 if width % 256 == 0 else 128)
    if exact:
        Mc = M
    else:
        Mc = M // 2 if M >= 512 else M
    nt = len(tps)

    args = [x2d]
    in_specs = [pl.BlockSpec((Mc, Cin), lambda n: (n, 0))]
    flops = 0
    for tp in tps:
        cin_t = tp["w1"].shape[0]
        args += [tp["w1"], tp["s1"], _pack_wb(tp["wb"], tile), tp["s2"],
                 tp["w3"], tp["s3"], tp["wr"], tp["sr"]]
        in_specs += [
            pl.BlockSpec((cin_t, width), lambda n: (0, 0)),
            pl.BlockSpec((1, width), lambda n: (0, 0)),
            pl.BlockSpec((3, width, tile), lambda n: (0, 0, 0)),
            pl.BlockSpec((1, width), lambda n: (0, 0)),
            pl.BlockSpec((width, Cout), lambda n: (0, 0)),
            pl.BlockSpec((1, Cout), lambda n: (0, 0)),
            pl.BlockSpec((cin_t, Cout), lambda n: (0, 0)),
            pl.BlockSpec((1, Cout), lambda n: (0, 0)),
        ]
        flops += 2 * M * (cin_t * width + 3 * tile * width
                          + width * Cout + cin_t * Cout)
    bytes_acc = sum(int(a.size) * a.dtype.itemsize for a in args) \
        + M * Cout * 2

    out = pl.pallas_call(
        functools.partial(_floor_body, L=L, width=width, tile=tile, nt=nt,
                          exact=exact),
        out_shape=jax.ShapeDtypeStruct((M, Cout), BF16),
        grid_spec=pltpu.PrefetchScalarGridSpec(
            num_scalar_prefetch=0,
            grid=(M // Mc,),
            in_specs=in_specs,
            out_specs=pl.BlockSpec((Mc, Cout), lambda n: (n, 0)),
            scratch_shapes=[pltpu.VMEM((Mc + 16, width), BF16)],
        ),
        compiler_params=pltpu.CompilerParams(
            dimension_semantics=("parallel",),
            vmem_limit_bytes=_VMEM),
        cost_estimate=pl.CostEstimate(flops=flops, transcendentals=0,
                                      bytes_accessed=bytes_acc),
    )(*args)
    return out.reshape(B, L, Cout)


def _seed_where_lk(y):
    return jnp.where(y > 0, y, _SLOPE * y)


def _seed_tenant_kernel(x_ref, w1_ref, s1_ref, wb_ref, s2_ref, w3_ref,
                        s3_ref, wr_ref, sr_ref, o_ref, hbuf_ref, *, L, width):
    M = x_ref.shape[0]
    x = x_ref[...]
    h1 = _seed_where_lk(jnp.dot(x, w1_ref[...], preferred_element_type=F32)
                        + s1_ref[...]).astype(BF16)
    hbuf_ref[...] = jnp.zeros_like(hbuf_ref)
    hbuf_ref[pl.ds(8, M), :] = h1
    row = jax.lax.broadcasted_iota(jnp.int32, (M, 1), 0)
    is_first = row == 0
    is_last = row == L - 1
    for b in range(1, M // L):
        is_first = jnp.logical_or(is_first, row == b * L)
        is_last = jnp.logical_or(is_last, row == b * L + L - 1)
    zeros = jnp.zeros((M, width), BF16)
    h_prev = jnp.where(is_first, zeros, hbuf_ref[pl.ds(7, M), :])
    h_next = jnp.where(is_last, zeros, hbuf_ref[pl.ds(9, M), :])
    s2 = s2_ref[...]
    wb0, wb1, wb2 = wb_ref[0], wb_ref[1], wb_ref[2]
    tn = o_ref.shape[1]
    acc = jnp.zeros((M, tn), F32)
    for j in range(width // 128):
        lo = j * 128
        g = (jnp.dot(h_prev[:, lo:lo + 128], wb0[lo:lo + 128, :],
                     preferred_element_type=F32)
             + jnp.dot(h1[:, lo:lo + 128], wb1[lo:lo + 128, :],
                       preferred_element_type=F32)
             + jnp.dot(h_next[:, lo:lo + 128], wb2[lo:lo + 128, :],
                       preferred_element_type=F32))
        h2 = _seed_where_lk(g + s2[:, lo:lo + 128]).astype(BF16)
        acc = acc + jnp.dot(h2, w3_ref[lo:lo + 128, :],
                            preferred_element_type=F32)
    idy = jnp.dot(x, wr_ref[...], preferred_element_type=F32) + sr_ref[...]
    o_ref[...] = _seed_where_lk(acc + s3_ref[...] + idy).astype(o_ref.dtype)


def _seed_tenant(x2d, tp, L):
    M, Cin = x2d.shape
    width = tp["w1"].shape[1]
    Cout = tp["w3"].shape[1]
    tn = min(512, Cout // 2)
    nsteps = Cout // tn
    fn = functools.partial(_seed_tenant_kernel, L=L, width=width)
    flops = 2 * M * ((Cin + 3 * 128) * width * nsteps
                     + width * Cout + Cin * Cout)
    bytes_accessed = 2 * (M * Cin + Cin * width + 3 * width * 128
                          + width * Cout + Cin * Cout + M * Cout) \
        + 4 * (2 * width + 2 * Cout)
    return pl.pallas_call(
        fn,
        out_shape=jax.ShapeDtypeStruct((M, Cout), BF16),
        grid_spec=pltpu.PrefetchScalarGridSpec(
            num_scalar_prefetch=0,
            grid=(nsteps,),
            in_specs=[
                pl.BlockSpec((M, Cin), lambda n: (0, 0)),
                pl.BlockSpec((Cin, width), lambda n: (0, 0)),
                pl.BlockSpec((1, width), lambda n: (0, 0)),
                pl.BlockSpec((3, width, 128), lambda n: (0, 0, 0)),
                pl.BlockSpec((1, width), lambda n: (0, 0)),
                pl.BlockSpec((width, tn), lambda n: (0, n)),
                pl.BlockSpec((1, tn), lambda n: (0, n)),
                pl.BlockSpec((Cin, tn), lambda n: (0, n)),
                pl.BlockSpec((1, tn), lambda n: (0, n)),
            ],
            out_specs=pl.BlockSpec((M, tn), lambda n: (0, n)),
            scratch_shapes=[pltpu.VMEM((M + 16, width), BF16)],
        ),
        compiler_params=pltpu.CompilerParams(
            dimension_semantics=("parallel",),
            vmem_limit_bytes=32 * 1024 * 1024),
        cost_estimate=pl.CostEstimate(flops=flops, transcendentals=0,
                                      bytes_accessed=bytes_accessed),
    )(x2d, tp["w1"], tp["s1"], tp["wb"], tp["s2"],
      tp["w3"], tp["s3"], tp["wr"], tp["sr"])


def _seed_floor(x3, tps):
    B, L, Cin = x3.shape
    h = x3.reshape(B * L, Cin)
    for tp in tps:
        h = _seed_tenant(h, tp, L)
    return h.reshape(B, L, -1)


def _seed_path_kernel(h_ref, sc_ref, sh_ref, w_ref, b_ref, o_ref, *, C):
    h = h_ref[...].astype(F32)
    pooled = 0.5 * (h[:, :C] + h[:, C:])
    a = _seed_where_lk(pooled * sc_ref[...] + sh_ref[...]).astype(BF16)
    y = jnp.dot(a, w_ref[...], preferred_element_type=F32) + b_ref[...]
    o_ref[...] = y.astype(o_ref.dtype)


def _seed_path(x3, pp):
    B, L, C = x3.shape
    Lo = L // 2
    h2 = x3[:, :Lo * 2, :].reshape(B * Lo, 2 * C)
    M = B * Lo
    tn = max(128, C // 2)
    out = pl.pallas_call(
        functools.partial(_seed_path_kernel, C=C),
        out_shape=jax.ShapeDtypeStruct((M, C), BF16),
        grid_spec=pltpu.PrefetchScalarGridSpec(
            num_scalar_prefetch=0,
            grid=(C // tn,),
            in_specs=[
                pl.BlockSpec((M, 2 * C), lambda n: (0, 0)),
                pl.BlockSpec((1, C), lambda n: (0, 0)),
                pl.BlockSpec((1, C), lambda n: (0, 0)),
                pl.BlockSpec((C, tn), lambda n: (0, n)),
                pl.BlockSpec((1, tn), lambda n: (0, n)),
            ],
            out_specs=pl.BlockSpec((M, tn), lambda n: (0, n))),
        compiler_params=pltpu.CompilerParams(
            dimension_semantics=("parallel",),
            vmem_limit_bytes=32 * 1024 * 1024),
    )(h2, pp["scale"], pp["shift"], pp["w"], pp["b"])
    return out.reshape(B, Lo, C)


def _pre_body(p0, p1, p2, w0, s0, q1, wi1, si1, q2, wi2, si2, q3, wi3, si3,
              o0, o1, o2, o3):
    w, s = w0[...], s0[...]
    y = None
    for p_ref in (p0, p1, p2):
        a = _lk(jnp.dot(p_ref[...], w, preferred_element_type=F32) + s)
        y = a if y is None else jnp.maximum(y, a)
    o0[...] = y.astype(BF16)
    for q, wi, si, o in ((q1, wi1, si1, o1), (q2, wi2, si2, o2),
                         (q3, wi3, si3, o3)):
        o[...] = (jnp.dot(q[...], wi[...], preferred_element_type=F32)
                  + si[...]).astype(BF16)


def _run_pre(parts, p0, q1, p1, q2, p2, q3, p3, B):
    shapes = (jax.ShapeDtypeStruct((B * 64, 64), BF16),
              jax.ShapeDtypeStruct((B * 64, 256), BF16),
              jax.ShapeDtypeStruct((B * 32, 512), BF16),
              jax.ShapeDtypeStruct((B * 16, 1024), BF16))
    args = (parts[0], parts[1], parts[2], p0["w"], p0["s"],
            q1, p1["w"], p1["s"], q2, p2["w"], p2["s"], q3, p3["w"], p3["s"])
    in_specs = []
    for a in args:
        if a.shape[0] in (1, 8):
            in_specs.append(pl.BlockSpec(a.shape, lambda n: (0, 0)))
        else:
            in_specs.append(pl.BlockSpec((a.shape[0] // 2, a.shape[1]),
                                         lambda n: (n, 0)))
    out_specs = [pl.BlockSpec((s.shape[0] // 2, s.shape[1]),
                              lambda n: (n, 0)) for s in shapes]
    return pl.pallas_call(
        _pre_body,
        out_shape=tuple(shapes),
        grid_spec=pltpu.PrefetchScalarGridSpec(
            num_scalar_prefetch=0, grid=(2,),
            in_specs=in_specs, out_specs=out_specs),
        compiler_params=pltpu.CompilerParams(
            dimension_semantics=("parallel",),
            vmem_limit_bytes=_VMEM),
    )(*args)


def _path_body(h_ref, sc_ref, sh_ref, w_ref, b_ref, o_ref, *, C):
    hv = h_ref[...].astype(F32)
    pooled = 0.5 * (hv[:, :C] + hv[:, C:])
    a = _lk(pooled * sc_ref[...] + sh_ref[...]).astype(BF16)
    o_ref[...] = (jnp.dot(a, w_ref[...], preferred_element_type=F32)
                  + b_ref[...]).astype(BF16)


def _run_path(hcat, pp):
    B, L2, C = hcat.shape
    pairs = hcat.reshape(B, L2 // 2, 2 * C)[:, ::2]
    Lo = pairs.shape[1]
    M = B * Lo
    h2 = pairs.reshape(M, 2 * C)
    out = pl.pallas_call(
        functools.partial(_path_body, C=C),
        out_shape=jax.ShapeDtypeStruct((M, C), BF16),
        grid_spec=pltpu.PrefetchScalarGridSpec(
            num_scalar_prefetch=0, grid=(2,),
            in_specs=[
                pl.BlockSpec((M // 2, 2 * C), lambda n: (n, 0)),
                pl.BlockSpec((1, C), lambda n: (0, 0)),
                pl.BlockSpec((1, C), lambda n: (0, 0)),
                pl.BlockSpec((C, C), lambda n: (0, 0)),
                pl.BlockSpec((1, C), lambda n: (0, 0)),
            ],
            out_specs=pl.BlockSpec((M // 2, C), lambda n: (n, 0))),
        compiler_params=pltpu.CompilerParams(
            dimension_semantics=("parallel",),
            vmem_limit_bytes=_VMEM),
    )(h2, pp["scale"], pp["shift"], pp["w"], pp["b"])
    return out.reshape(B, Lo, C)


def _head_body(h_ref, w1_ref, b1_ref, w2_ref, b2_ref, o_ref):
    pooled = jnp.mean(h_ref[...].astype(F32), axis=1)
    a = _lk(jnp.dot(pooled.astype(BF16), w1_ref[...],
                    preferred_element_type=F32) + b1_ref[...])
    o_ref[...] = jnp.dot(a.astype(BF16), w2_ref[...],
                         preferred_element_type=F32) + b2_ref[...]


def _run_head(h, fc1, fc2):
    B = h.shape[0]
    label = fc2["w"].shape[1]
    return pl.pallas_call(
        _head_body,
        out_shape=jax.ShapeDtypeStruct((B, label), F32),
        in_specs=[pl.BlockSpec(memory_space=pltpu.MemorySpace.VMEM)] * 5,
        out_specs=pl.BlockSpec(memory_space=pltpu.MemorySpace.VMEM),
    )(h, fc1["w"], fc1["b"], fc2["w"], fc2["b"])


def _forward(p, x):
    B = x.shape[0]
    xs = x[:, :, 0]
    taps = jnp.stack([xs[:, t:t + 192] for t in range(7)], axis=-1)
    taps8 = jnp.pad(taps, ((0, 0), (0, 0), (0, 1)))
    parts = [taps8[:, j::3, :].reshape(B * 64, 8).astype(BF16)
             for j in range(3)]
    pat = jnp.concatenate([taps, jnp.ones((B, 192, 1), F32)], axis=-1)
    q1 = pat.reshape(B * 64, 3, 8).mean(axis=1)
    q2 = pat.reshape(B * 32, 6, 8).mean(axis=1)
    xs3 = jnp.pad(xs, ((0, 0), (2, 2)))
    taps3 = jnp.stack([xs3[:, t:t + 196] for t in range(7)], axis=-1)
    pat3 = jnp.pad(jnp.concatenate([taps3, jnp.ones((B, 196, 1), F32)],
                                   axis=-1), ((0, 0), (2, 2), (0, 0)))
    q3 = pat3[:, :192, :].reshape(B * 16, 12, 8).mean(axis=1)

    h0, id1f, id2f, id3f = _run_pre(parts, p["stage0"], q1, p["stage1_"],
                                    q2, p["stage2_"], q3, p["stage3_"], B)
    h = h0.reshape(B, 64, 64)
    id1 = id1f.reshape(B, 64, 256)
    id2 = id2f.reshape(B, 32, 512)
    id3 = id3f.reshape(B, 16, 1024)
    id1e, id2e, id3e = id1[:, ::2], id2[:, ::2], id3[:, ::2]

    h = _run_floor(h, p["stage1_1"], exact=True)
    for nm in ("stage1_2", "stage1_3", "stage1_4"):
        h = _run_floor(jnp.concatenate([h[:, ::2], id1e], axis=1), p[nm],
                       exact=True)
    h = _seed_path(jnp.concatenate([h, id1], axis=1), p["layer1_path"])

    h = _seed_floor(h[:, ::2], p["stage2_1"])
    for nm in ("stage2_2", "stage2_3", "stage2_4"):
        h = _seed_floor(jnp.concatenate([h[:, ::2], id2e], axis=1), p[nm])
    h = _seed_path(jnp.concatenate([h, id2], axis=1), p["layer2_path"])

    h = _seed_floor(h[:, ::2], p["stage3_1"])
    for nm in ("stage3_2", "stage3_3", "stage3_4"):
        h = _seed_floor(jnp.concatenate([h[:, ::2], id3e], axis=1), p[nm])
    h = _seed_path(jnp.concatenate([h, id3], axis=1), p["layer3_path"])

    h = _run_floor(h[:, ::2], p["stage4"])
    return _run_head(h, p["fc1"], p["fc2"])


def kernel(arr0000, arr0001, arr0002, arr0003, arr0004, arr0005, arr0006, arr0007, arr0008, arr0009, arr0010, arr0011, arr0012, arr0013, arr0014, arr0015, arr0016, arr0017, arr0018, arr0019, arr0020, arr0021, arr0022, arr0023, arr0024, arr0025, arr0026, arr0027, arr0028, arr0029, arr0030, arr0031, arr0032, arr0033, arr0034, arr0035, arr0036, arr0037, arr0038, arr0039, arr0040, arr0041, arr0042, arr0043, arr0044, arr0045, arr0046, arr0047, arr0048, arr0049, arr0050, arr0051, arr0052, arr0053, arr0054, arr0055, arr0056, arr0057, arr0058, arr0059, arr0060, arr0061, arr0062, arr0063, arr0064, arr0065, arr0066, arr0067, arr0068, arr0069, arr0070, arr0071, arr0072, arr0073, arr0074, arr0075, arr0076, arr0077, arr0078, arr0079, arr0080, arr0081, arr0082, arr0083, arr0084, arr0085, arr0086, arr0087, arr0088, arr0089, arr0090, arr0091, arr0092, arr0093, arr0094, arr0095, arr0096, arr0097, arr0098, arr0099, arr0100, arr0101, arr0102, arr0103, arr0104, arr0105, arr0106, arr0107, arr0108, arr0109, arr0110, arr0111, arr0112, arr0113, arr0114, arr0115, arr0116, arr0117, arr0118, arr0119, arr0120, arr0121, arr0122, arr0123, arr0124, arr0125, arr0126, arr0127, arr0128, arr0129, arr0130, arr0131, arr0132, arr0133, arr0134, arr0135, arr0136, arr0137, arr0138, arr0139, arr0140, arr0141, arr0142, arr0143, arr0144, arr0145, arr0146, arr0147, arr0148, arr0149, arr0150, arr0151, arr0152, arr0153, arr0154, arr0155, arr0156, arr0157, arr0158, arr0159, arr0160, arr0161, arr0162, arr0163, arr0164, arr0165, arr0166, arr0167, arr0168, arr0169, arr0170, arr0171, arr0172, arr0173, arr0174, arr0175, arr0176, arr0177, arr0178, arr0179, arr0180, arr0181, arr0182, arr0183, arr0184, arr0185, arr0186, arr0187, arr0188, arr0189, arr0190, arr0191, arr0192, arr0193, arr0194, arr0195, arr0196, arr0197, arr0198, arr0199, arr0200, arr0201, arr0202, arr0203, arr0204, arr0205, arr0206, arr0207, arr0208, arr0209, arr0210, arr0211, arr0212, arr0213, arr0214, arr0215, arr0216, arr0217, arr0218, arr0219, arr0220, arr0221, arr0222, arr0223, arr0224, arr0225, arr0226, arr0227, arr0228, arr0229, arr0230, arr0231, arr0232, arr0233, arr0234, arr0235, arr0236, arr0237, arr0238, arr0239, arr0240, arr0241, arr0242, arr0243, arr0244, arr0245, arr0246, arr0247, arr0248, arr0249, arr0250, arr0251, arr0252, arr0253, arr0254, arr0255, arr0256, arr0257, arr0258, arr0259, arr0260, arr0261, arr0262, arr0263, arr0264, arr0265, arr0266, arr0267, arr0268, arr0269, arr0270, arr0271, arr0272, arr0273, arr0274, arr0275, arr0276, arr0277, arr0278, arr0279, arr0280, arr0281, arr0282, arr0283, arr0284, arr0285, arr0286, arr0287, arr0288, arr0289, arr0290, arr0291, arr0292, arr0293, arr0294, arr0295, arr0296, arr0297, arr0298, arr0299, arr0300, arr0301, arr0302, arr0303, arr0304, arr0305, arr0306, arr0307, arr0308, arr0309, arr0310, arr0311, arr0312, arr0313, arr0314, arr0315, arr0316, arr0317, arr0318, arr0319, arr0320, arr0321, arr0322, arr0323, arr0324, arr0325, arr0326, arr0327, arr0328, arr0329, arr0330, arr0331, arr0332, arr0333, arr0334, arr0335, arr0336):
    flat = list(locals().values())
    params, x = jax.tree_util.tree_unflatten(_TREEDEF, flat)
    return _forward(params, x)
